# Initial kernel scaffold; baseline (speedup 1.0000x reference)
import jax
import jax.numpy as jnp
from jax.experimental import pallas as pl


def kernel(x, edge_index, theta):
    raise NotImplementedError("write your pallas kernel here")


# speedup vs baseline: 107.6743x; 107.6743x over previous
import functools

import jax
import jax.numpy as jnp
from jax import lax
from jax.experimental import pallas as pl
from jax.experimental.pallas import tpu as pltpu
from jax.experimental.pallas import tpu_sc as plsc

_N = 100000
_E = 1600000
_T = 10
_L = 12
_NTILES = 16
_NPAD = 100096
_SL = _NPAD // _NTILES
_NV = _SL // 16
_CE = 12800
_EPT = 102400
_NCHUNK = _EPT // _CE
_EPAD = _EPT * _NTILES


def _sc_body(src_hbm, dst_hbm, x_hbm, scale_hbm, out_hbm,
             q_sh, ypos_sh, yneg_sh,
             srcb, dstb, qs, qd,
             dinv2_v, sdeg_v, pacc_v, ytp, ytn, zeros_v, scale_v):
    wid = lax.axis_index("s")
    base_n = wid * _SL
    base_e = wid * _EPT
    nsl = pl.ds(base_n, _SL)

    pltpu.sync_copy(scale_hbm, scale_v)

    def _zero_loop(j, c):
        zeros_v[pl.ds(j * 16, 16)] = jnp.zeros((16,), jnp.float32)
        return c
    lax.fori_loop(0, _NV, _zero_loop, 0)

    def _ones_loop(j, c):
        qs[pl.ds(j * 16, 16)] = jnp.ones((16,), jnp.float32)
        return c
    lax.fori_loop(0, _CE // 16, _ones_loop, 0)

    pltpu.sync_copy(zeros_v, ypos_sh.at[nsl])
    pltpu.sync_copy(zeros_v, yneg_sh.at[nsl])
    plsc.subcore_barrier()

    def _deg_loop(i, c):
        pltpu.sync_copy(dst_hbm.at[pl.ds(base_e + i * _CE, _CE)], dstb)
        pltpu.sync_copy(qs, ypos_sh.at[dstb], add=True)
        return c
    lax.fori_loop(0, _NCHUNK, _deg_loop, 0)
    plsc.subcore_barrier()

    pltpu.sync_copy(ypos_sh.at[nsl], ytp)
    pltpu.sync_copy(x_hbm.at[nsl], ytn)

    def _init_loop(j, c):
        s = pl.ds(j * 16, 16)
        deg16 = ytp[s] + 1.0
        dinv2_v[s] = 1.0 / deg16
        bits = lax.bitcast_convert_type(deg16, jnp.int32)
        y = lax.bitcast_convert_type(jnp.int32(0x5F3759DF) - (bits >> 1),
                                     jnp.float32)
        for _ in range(4):
            y = y * (1.5 - 0.5 * deg16 * y * y)
        sdeg_v[s] = deg16 * y
        p0 = ytn[s] * y
        pacc_v[s] = p0
        ytn[s] = p0
        return c
    lax.fori_loop(0, _NV, _init_loop, 0)

    pltpu.sync_copy(ytn, q_sh.at[nsl])
    pltpu.sync_copy(zeros_v, ypos_sh.at[nsl])
    plsc.subcore_barrier()

    def _app_loop(a, c):
        sc16 = plsc.load_gather(scale_v, [jnp.full((16,), a, dtype=jnp.int32)])

        def _edge_loop(i, cc):
            r = pl.ds(base_e + i * _CE, _CE)
            pltpu.sync_copy(src_hbm.at[r], srcb)
            pltpu.sync_copy(dst_hbm.at[r], dstb)
            pltpu.sync_copy(q_sh.at[srcb], qs)
            pltpu.sync_copy(qs, ypos_sh.at[dstb], add=True)
            pltpu.sync_copy(q_sh.at[dstb], qd)
            pltpu.sync_copy(qd, yneg_sh.at[srcb], add=True)
            return cc
        lax.fori_loop(0, _NCHUNK, _edge_loop, 0)
        plsc.subcore_barrier()

        pltpu.sync_copy(ypos_sh.at[nsl], ytp)
        pltpu.sync_copy(yneg_sh.at[nsl], ytn)
        pltpu.sync_copy(zeros_v, ypos_sh.at[nsl])
        pltpu.sync_copy(zeros_v, yneg_sh.at[nsl])

        is_end16 = jnp.full((16,), a % _T, jnp.int32) == (_T - 1)

        def _pw_loop(j, cc):
            s = pl.ds(j * 16, 16)
            p16 = sc16 * dinv2_v[s] * (ytp[s] - ytn[s])
            acc16 = pacc_v[s] + p16
            pacc_v[s] = acc16
            ytp[s] = jnp.where(is_end16, acc16, p16)
            return cc
        lax.fori_loop(0, _NV, _pw_loop, 0)

        pltpu.sync_copy(ytp, q_sh.at[nsl])
        plsc.subcore_barrier()
        return c
    lax.fori_loop(0, _L * _T, _app_loop, 0)

    def _out_loop(j, c):
        s = pl.ds(j * 16, 16)
        ytp[s] = sdeg_v[s] * pacc_v[s]
        return c
    lax.fori_loop(0, _NV, _out_loop, 0)
    pltpu.sync_copy(ytp, out_hbm.at[nsl])


_sc_call = functools.partial(
    pl.kernel,
    out_type=jax.ShapeDtypeStruct((_NPAD,), jnp.float32),
    mesh=plsc.VectorSubcoreMesh(
        core_axis_name="c", subcore_axis_name="s", num_cores=1),
    compiler_params=pltpu.CompilerParams(needs_layout_passes=False),
    scratch_types=[
        pltpu.VMEM_SHARED((_NPAD,), jnp.float32),
        pltpu.VMEM_SHARED((_NPAD,), jnp.float32),
        pltpu.VMEM_SHARED((_NPAD,), jnp.float32),
        pltpu.VMEM((_CE,), jnp.int32),
        pltpu.VMEM((_CE,), jnp.int32),
        pltpu.VMEM((_CE,), jnp.float32),
        pltpu.VMEM((_CE,), jnp.float32),
        pltpu.VMEM((_SL,), jnp.float32),
        pltpu.VMEM((_SL,), jnp.float32),
        pltpu.VMEM((_SL,), jnp.float32),
        pltpu.VMEM((_SL,), jnp.float32),
        pltpu.VMEM((_SL,), jnp.float32),
        pltpu.VMEM((_SL,), jnp.float32),
        pltpu.VMEM((128,), jnp.float32),
    ],
)(_sc_body)


def kernel(x, edge_index, theta):
    xf = jnp.pad(x.reshape(_N), (0, _NPAD - _N))
    padi = jnp.full((_EPAD - _E,), _NPAD - 1, jnp.int32)
    src2 = jnp.concatenate([edge_index[0], padi])
    dst2 = jnp.concatenate([edge_index[1], padi])
    a_idx = jnp.arange(_L * _T)
    scales = theta[a_idx // _T] / (a_idx % _T + 1).astype(jnp.float32)
    scales = jnp.pad(scales, (0, 128 - _L * _T))
    out = _sc_call(src2, dst2, xf, scales)
    return out[:_N].reshape(_N, 1, 1)


# speedup vs baseline: 140.2092x; 1.3022x over previous
import functools

import jax
import jax.numpy as jnp
from jax import lax
from jax.experimental import pallas as pl
from jax.experimental.pallas import tpu as pltpu
from jax.experimental.pallas import tpu_sc as plsc

_N = 100000
_E = 1600000
_T = 10
_L = 12
_NTILES = 16
_NPAD = 100096
_SL = _NPAD // _NTILES
_NV = _SL // 16
_CE = 6400
_EPT = 102400
_NCHUNK = _EPT // _CE
_EPAD = _EPT * _NTILES


def _sc_body(src_hbm, dst_hbm, x_hbm, scale_hbm, out_hbm,
             q_sh, ypos_sh, yneg_sh,
             srcb0, srcb1, srcb2, dstb0, dstb1, dstb2, qs0, qs1, qd0, qd1,
             dinv2_v, sdeg_v, pacc_v, ytp, ytn, zeros_v, scale_v,
             sem_ld0, sem_ld1, sem_ld2, sem_g0, sem_g1, sem_s0, sem_s1,
             sem_p0, sem_p1, sem_p2):
    wid = lax.axis_index("s")
    base_n = wid * _SL
    base_e = wid * _EPT
    nsl = pl.ds(base_n, _SL)
    srcb = (srcb0, srcb1, srcb2)
    dstb = (dstb0, dstb1, dstb2)
    qs = (qs0, qs1)
    qd = (qd0, qd1)
    sem_ld = (sem_ld0, sem_ld1, sem_ld2)
    sem_g = (sem_g0, sem_g1)
    sem_s = (sem_s0, sem_s1)

    pltpu.sync_copy(scale_hbm, scale_v)

    def _zero_loop(j, c):
        zeros_v[pl.ds(j * 16, 16)] = jnp.zeros((16,), jnp.float32)
        return c
    lax.fori_loop(0, _NV, _zero_loop, 0)

    def _ones_loop(j, c):
        qs0[pl.ds(j * 16, 16)] = jnp.ones((16,), jnp.float32)
        return c
    lax.fori_loop(0, _CE // 16, _ones_loop, 0)

    pltpu.sync_copy(zeros_v, ypos_sh.at[nsl])
    pltpu.sync_copy(zeros_v, yneg_sh.at[nsl])
    plsc.subcore_barrier()

    def _deg_loop(i, c):
        pltpu.sync_copy(dst_hbm.at[pl.ds(base_e + i * _CE, _CE)], dstb0)
        pltpu.sync_copy(qs0, ypos_sh.at[dstb0], add=True)
        return c
    lax.fori_loop(0, _NCHUNK, _deg_loop, 0)
    plsc.subcore_barrier()

    pltpu.sync_copy(ypos_sh.at[nsl], ytp)
    pltpu.sync_copy(x_hbm.at[nsl], ytn)

    def _init_loop(j, c):
        s = pl.ds(j * 16, 16)
        deg16 = ytp[s] + 1.0
        dinv2_v[s] = 1.0 / deg16
        bits = lax.bitcast_convert_type(deg16, jnp.int32)
        y = lax.bitcast_convert_type(jnp.int32(0x5F3759DF) - (bits >> 1),
                                     jnp.float32)
        for _ in range(4):
            y = y * (1.5 - 0.5 * deg16 * y * y)
        sdeg_v[s] = deg16 * y
        p0 = ytn[s] * y
        pacc_v[s] = p0
        ytn[s] = p0
        return c
    lax.fori_loop(0, _NV, _init_loop, 0)

    pltpu.sync_copy(ytn, q_sh.at[nsl])
    pltpu.sync_copy(zeros_v, ypos_sh.at[nsl])
    plsc.subcore_barrier()

    def _app_loop(a, c):
        sc16 = plsc.load_gather(scale_v, [jnp.full((16,), a, dtype=jnp.int32)])

        def _fire_ld(i):
            bi = i % 3
            r = pl.ds(base_e + i * _CE, _CE)
            d1 = pltpu.async_copy(src_hbm.at[r], srcb[bi], sem_ld[bi])
            d2 = pltpu.async_copy(dst_hbm.at[r], dstb[bi], sem_ld[bi])
            return (d1, d2)

        ld_desc = {0: _fire_ld(0)}
        sc_desc = {}
        for i in range(_NCHUNK):
            bi = i % 3
            bv = i % 2
            if i >= 2:
                sc_desc[i - 2][0].wait()
                sc_desc[i - 2][1].wait()
            if i + 1 < _NCHUNK:
                ld_desc[i + 1] = _fire_ld(i + 1)
            ld_desc[i][0].wait()
            ld_desc[i][1].wait()
            g1 = pltpu.async_copy(q_sh.at[srcb[bi]], qs[bv], sem_g[bv])
            g2 = pltpu.async_copy(q_sh.at[dstb[bi]], qd[bv], sem_g[bv])
            g1.wait()
            g2.wait()
            s1 = pltpu.async_copy(qs[bv], ypos_sh.at[dstb[bi]], sem_s[bv],
                                  add=True)
            s2 = pltpu.async_copy(qd[bv], yneg_sh.at[srcb[bi]], sem_s[bv],
                                  add=True)
            sc_desc[i] = (s1, s2)
        for i in (_NCHUNK - 2, _NCHUNK - 1):
            sc_desc[i][0].wait()
            sc_desc[i][1].wait()
        plsc.subcore_barrier()

        da = pltpu.async_copy(ypos_sh.at[nsl], ytp, sem_p0)
        db = pltpu.async_copy(yneg_sh.at[nsl], ytn, sem_p1)
        da.wait()
        db.wait()
        z1 = pltpu.async_copy(zeros_v, ypos_sh.at[nsl], sem_p0)
        z2 = pltpu.async_copy(zeros_v, yneg_sh.at[nsl], sem_p1)

        is_end16 = jnp.full((16,), a % _T, jnp.int32) == (_T - 1)

        def _pw_loop(j, cc):
            s = pl.ds(j * 16, 16)
            p16 = sc16 * dinv2_v[s] * (ytp[s] - ytn[s])
            acc16 = pacc_v[s] + p16
            pacc_v[s] = acc16
            ytp[s] = jnp.where(is_end16, acc16, p16)
            return cc
        lax.fori_loop(0, _NV, _pw_loop, 0)

        wq = pltpu.async_copy(ytp, q_sh.at[nsl], sem_p2)
        z1.wait()
        z2.wait()
        wq.wait()
        plsc.subcore_barrier()
        return c
    lax.fori_loop(0, _L * _T, _app_loop, 0)

    def _out_loop(j, c):
        s = pl.ds(j * 16, 16)
        ytp[s] = sdeg_v[s] * pacc_v[s]
        return c
    lax.fori_loop(0, _NV, _out_loop, 0)
    pltpu.sync_copy(ytp, out_hbm.at[nsl])


_sc_call = functools.partial(
    pl.kernel,
    out_type=jax.ShapeDtypeStruct((_NPAD,), jnp.float32),
    mesh=plsc.VectorSubcoreMesh(
        core_axis_name="c", subcore_axis_name="s", num_cores=1),
    compiler_params=pltpu.CompilerParams(needs_layout_passes=False),
    scratch_types=[
        pltpu.VMEM_SHARED((_NPAD,), jnp.float32),
        pltpu.VMEM_SHARED((_NPAD,), jnp.float32),
        pltpu.VMEM_SHARED((_NPAD,), jnp.float32),
        pltpu.VMEM((_CE,), jnp.int32),
        pltpu.VMEM((_CE,), jnp.int32),
        pltpu.VMEM((_CE,), jnp.int32),
        pltpu.VMEM((_CE,), jnp.int32),
        pltpu.VMEM((_CE,), jnp.int32),
        pltpu.VMEM((_CE,), jnp.int32),
        pltpu.VMEM((_CE,), jnp.float32),
        pltpu.VMEM((_CE,), jnp.float32),
        pltpu.VMEM((_CE,), jnp.float32),
        pltpu.VMEM((_CE,), jnp.float32),
        pltpu.VMEM((_SL,), jnp.float32),
        pltpu.VMEM((_SL,), jnp.float32),
        pltpu.VMEM((_SL,), jnp.float32),
        pltpu.VMEM((_SL,), jnp.float32),
        pltpu.VMEM((_SL,), jnp.float32),
        pltpu.VMEM((_SL,), jnp.float32),
        pltpu.VMEM((128,), jnp.float32),
        pltpu.SemaphoreType.DMA,
        pltpu.SemaphoreType.DMA,
        pltpu.SemaphoreType.DMA,
        pltpu.SemaphoreType.DMA,
        pltpu.SemaphoreType.DMA,
        pltpu.SemaphoreType.DMA,
        pltpu.SemaphoreType.DMA,
        pltpu.SemaphoreType.DMA,
        pltpu.SemaphoreType.DMA,
        pltpu.SemaphoreType.DMA,
    ],
)(_sc_body)


def kernel(x, edge_index, theta):
    xf = jnp.pad(x.reshape(_N), (0, _NPAD - _N))
    padi = jnp.full((_EPAD - _E,), _NPAD - 1, jnp.int32)
    src2 = jnp.concatenate([edge_index[0], padi])
    dst2 = jnp.concatenate([edge_index[1], padi])
    a_idx = jnp.arange(_L * _T)
    scales = theta[a_idx // _T] / (a_idx % _T + 1).astype(jnp.float32)
    scales = jnp.pad(scales, (0, 128 - _L * _T))
    out = _sc_call(src2, dst2, xf, scales)
    return out[:_N].reshape(_N, 1, 1)


# speedup vs baseline: 198.0014x; 1.4122x over previous
import functools

import jax
import jax.numpy as jnp
from jax import lax
from jax.experimental import pallas as pl
from jax.experimental.pallas import tpu as pltpu
from jax.experimental.pallas import tpu_sc as plsc

_N = 100000
_E = 1600000
_T = 10
_L = 12
_NTILES = 16
_NCORES = 2
_NPAD = 100096
_SL = _NPAD // _NTILES
_NV = _SL // 16
_CE = 6400
_EPT = 51200
_NCHUNK = _EPT // _CE
_EPAD = _EPT * _NTILES * _NCORES


def _sc_body(src_hbm, dst_hbm, x_hbm, scale_hbm, out_hbm, xdiff_hbm,
             q_sh, ypos_sh, yneg_sh,
             srcb0, srcb1, srcb2, dstb0, dstb1, dstb2, qs0, qs1, qd0, qd1,
             dinv2_v, sdeg_v, pacc_v, ytp, ytn, zeros_v, scale_v,
             sem_ld0, sem_ld1, sem_ld2, sem_g0, sem_g1, sem_s0, sem_s1,
             sem_p0, sem_p1, sem_p2, sem_x):
    cid = lax.axis_index("c")
    wid = lax.axis_index("s")
    base_n = wid * _SL
    base_e = (cid * _NTILES + wid) * _EPT
    nsl = pl.ds(base_n, _SL)
    own_x = pl.ds(cid * _NPAD + base_n, _SL)
    mir_x = pl.ds((1 - cid) * _NPAD + base_n, _SL)
    srcb = (srcb0, srcb1, srcb2)
    dstb = (dstb0, dstb1, dstb2)
    qs = (qs0, qs1)
    qd = (qd0, qd1)
    sem_ld = (sem_ld0, sem_ld1, sem_ld2)
    sem_g = (sem_g0, sem_g1)
    sem_s = (sem_s0, sem_s1)

    def _cross_barrier():
        plsc.subcore_barrier()
        pl.semaphore_signal(sem_x, 1, core_index=1 - cid)
        pl.semaphore_wait(sem_x, 1)

    pltpu.sync_copy(scale_hbm, scale_v)

    def _zero_loop(j, c):
        zeros_v[pl.ds(j * 16, 16)] = jnp.zeros((16,), jnp.float32)
        return c
    lax.fori_loop(0, _NV, _zero_loop, 0)

    def _ones_loop(j, c):
        qs0[pl.ds(j * 16, 16)] = jnp.ones((16,), jnp.float32)
        return c
    lax.fori_loop(0, _CE // 16, _ones_loop, 0)

    pltpu.sync_copy(zeros_v, ypos_sh.at[nsl])
    pltpu.sync_copy(zeros_v, yneg_sh.at[nsl])
    plsc.subcore_barrier()

    def _deg_loop(i, c):
        pltpu.sync_copy(dst_hbm.at[pl.ds(base_e + i * _CE, _CE)], dstb0)
        pltpu.sync_copy(qs0, ypos_sh.at[dstb0], add=True)
        return c
    lax.fori_loop(0, _NCHUNK, _deg_loop, 0)
    plsc.subcore_barrier()

    pltpu.sync_copy(ypos_sh.at[nsl], ytp)
    pltpu.sync_copy(ytp, xdiff_hbm.at[own_x])
    pltpu.sync_copy(zeros_v, ypos_sh.at[nsl])
    _cross_barrier()
    pltpu.sync_copy(xdiff_hbm.at[mir_x], ytn)
    pltpu.sync_copy(x_hbm.at[nsl], qs1.at[pl.ds(0, _SL)])

    def _init_loop(j, c):
        s = pl.ds(j * 16, 16)
        deg16 = ytp[s] + ytn[s] + 1.0
        dinv2_v[s] = 1.0 / deg16
        bits = lax.bitcast_convert_type(deg16, jnp.int32)
        y = lax.bitcast_convert_type(jnp.int32(0x5F3759DF) - (bits >> 1),
                                     jnp.float32)
        for _ in range(4):
            y = y * (1.5 - 0.5 * deg16 * y * y)
        sdeg_v[s] = deg16 * y
        p0 = qs1[s] * y
        pacc_v[s] = p0
        ytn[s] = p0
        return c
    lax.fori_loop(0, _NV, _init_loop, 0)

    pltpu.sync_copy(ytn, q_sh.at[nsl])
    plsc.subcore_barrier()

    def _app_loop(a, c):
        sc16 = plsc.load_gather(scale_v, [jnp.full((16,), a, dtype=jnp.int32)])

        def _fire_ld(i):
            bi = i % 3
            r = pl.ds(base_e + i * _CE, _CE)
            d1 = pltpu.async_copy(src_hbm.at[r], srcb[bi], sem_ld[bi])
            d2 = pltpu.async_copy(dst_hbm.at[r], dstb[bi], sem_ld[bi])
            return (d1, d2)

        ld_desc = {0: _fire_ld(0)}
        sc_desc = {}
        for i in range(_NCHUNK):
            bi = i % 3
            bv = i % 2
            if i >= 2:
                sc_desc[i - 2][0].wait()
                sc_desc[i - 2][1].wait()
            if i + 1 < _NCHUNK:
                ld_desc[i + 1] = _fire_ld(i + 1)
            ld_desc[i][0].wait()
            ld_desc[i][1].wait()
            g1 = pltpu.async_copy(q_sh.at[srcb[bi]], qs[bv], sem_g[bv])
            g2 = pltpu.async_copy(q_sh.at[dstb[bi]], qd[bv], sem_g[bv])
            g1.wait()
            g2.wait()
            s1 = pltpu.async_copy(qs[bv], ypos_sh.at[dstb[bi]], sem_s[bv],
                                  add=True)
            s2 = pltpu.async_copy(qd[bv], yneg_sh.at[srcb[bi]], sem_s[bv],
                                  add=True)
            sc_desc[i] = (s1, s2)
        for i in (_NCHUNK - 2, _NCHUNK - 1):
            sc_desc[i][0].wait()
            sc_desc[i][1].wait()
        plsc.subcore_barrier()

        da = pltpu.async_copy(ypos_sh.at[nsl], ytp, sem_p0)
        db = pltpu.async_copy(yneg_sh.at[nsl], ytn, sem_p1)
        da.wait()
        db.wait()
        z1 = pltpu.async_copy(zeros_v, ypos_sh.at[nsl], sem_p0)
        z2 = pltpu.async_copy(zeros_v, yneg_sh.at[nsl], sem_p1)

        def _diff_loop(j, cc):
            s = pl.ds(j * 16, 16)
            ytp[s] = ytp[s] - ytn[s]
            return cc
        lax.fori_loop(0, _NV, _diff_loop, 0)

        xw = pltpu.async_copy(ytp, xdiff_hbm.at[own_x], sem_p2)
        xw.wait()
        _cross_barrier()
        xr = pltpu.async_copy(xdiff_hbm.at[mir_x], ytn, sem_p2)
        xr.wait()

        is_end16 = jnp.full((16,), a % _T, jnp.int32) == (_T - 1)

        def _pw_loop(j, cc):
            s = pl.ds(j * 16, 16)
            p16 = sc16 * dinv2_v[s] * (ytp[s] + ytn[s])
            acc16 = pacc_v[s] + p16
            pacc_v[s] = acc16
            ytp[s] = jnp.where(is_end16, acc16, p16)
            return cc
        lax.fori_loop(0, _NV, _pw_loop, 0)

        wq = pltpu.async_copy(ytp, q_sh.at[nsl], sem_p2)
        z1.wait()
        z2.wait()
        wq.wait()
        plsc.subcore_barrier()
        return c
    lax.fori_loop(0, _L * _T, _app_loop, 0)

    def _out_loop(j, c):
        s = pl.ds(j * 16, 16)
        ytp[s] = sdeg_v[s] * pacc_v[s]
        return c
    lax.fori_loop(0, _NV, _out_loop, 0)

    @pl.when(cid == 0)
    def _():
        pltpu.sync_copy(ytp, out_hbm.at[nsl])


_sc_call = functools.partial(
    pl.kernel,
    out_type=(jax.ShapeDtypeStruct((_NPAD,), jnp.float32),
              jax.ShapeDtypeStruct((_NCORES * _NPAD,), jnp.float32)),
    mesh=plsc.VectorSubcoreMesh(
        core_axis_name="c", subcore_axis_name="s", num_cores=_NCORES),
    compiler_params=pltpu.CompilerParams(needs_layout_passes=False),
    scratch_types=[
        pltpu.VMEM_SHARED((_NPAD,), jnp.float32),
        pltpu.VMEM_SHARED((_NPAD,), jnp.float32),
        pltpu.VMEM_SHARED((_NPAD,), jnp.float32),
        pltpu.VMEM((_CE,), jnp.int32),
        pltpu.VMEM((_CE,), jnp.int32),
        pltpu.VMEM((_CE,), jnp.int32),
        pltpu.VMEM((_CE,), jnp.int32),
        pltpu.VMEM((_CE,), jnp.int32),
        pltpu.VMEM((_CE,), jnp.int32),
        pltpu.VMEM((_CE,), jnp.float32),
        pltpu.VMEM((_CE,), jnp.float32),
        pltpu.VMEM((_CE,), jnp.float32),
        pltpu.VMEM((_CE,), jnp.float32),
        pltpu.VMEM((_SL,), jnp.float32),
        pltpu.VMEM((_SL,), jnp.float32),
        pltpu.VMEM((_SL,), jnp.float32),
        pltpu.VMEM((_SL,), jnp.float32),
        pltpu.VMEM((_SL,), jnp.float32),
        pltpu.VMEM((_SL,), jnp.float32),
        pltpu.VMEM((128,), jnp.float32),
        pltpu.SemaphoreType.DMA,
        pltpu.SemaphoreType.DMA,
        pltpu.SemaphoreType.DMA,
        pltpu.SemaphoreType.DMA,
        pltpu.SemaphoreType.DMA,
        pltpu.SemaphoreType.DMA,
        pltpu.SemaphoreType.DMA,
        pltpu.SemaphoreType.DMA,
        pltpu.SemaphoreType.DMA,
        pltpu.SemaphoreType.DMA,
        pltpu.SemaphoreType.REGULAR,
    ],
)(_sc_body)


def kernel(x, edge_index, theta):
    xf = jnp.pad(x.reshape(_N), (0, _NPAD - _N))
    padi = jnp.full((_EPAD - _E,), _NPAD - 1, jnp.int32)
    src2 = jnp.concatenate([edge_index[0], padi])
    dst2 = jnp.concatenate([edge_index[1], padi])
    a_idx = jnp.arange(_L * _T)
    scales = theta[a_idx // _T] / (a_idx % _T + 1).astype(jnp.float32)
    scales = jnp.pad(scales, (0, 128 - _L * _T))
    out, _ = _sc_call(src2, dst2, xf, scales)
    return out[:_N].reshape(_N, 1, 1)


# speedup vs baseline: 200.2676x; 1.0114x over previous
import functools

import jax
import jax.numpy as jnp
from jax import lax
from jax.experimental import pallas as pl
from jax.experimental.pallas import tpu as pltpu
from jax.experimental.pallas import tpu_sc as plsc

_N = 100000
_E = 1600000
_T = 10
_L = 12
_NTILES = 16
_NCORES = 2
_NPAD = 100096
_SL = _NPAD // _NTILES
_NV = _SL // 16
_CE = 6400
_EPT = 51200
_NCHUNK = _EPT // _CE
_EPAD = _EPT * _NTILES * _NCORES


def _sc_body(src_hbm, dst_hbm, x_hbm, scale_hbm, out_hbm, xdiff_hbm,
             q_sh, ypos_sh, yneg_sh,
             srcb0, srcb1, srcb2, dstb0, dstb1, dstb2, qs0, qs1, qd0, qd1,
             dinv2_v, sdeg_v, pacc_v, ytp, ytn, zeros_v, scale_v,
             sem_ld0, sem_ld1, sem_ld2, sem_g0, sem_g1, sem_s0, sem_s1,
             sem_p0, sem_p1, sem_p2, sem_x):
    cid = lax.axis_index("c")
    wid = lax.axis_index("s")
    base_n = wid * _SL
    base_e = (cid * _NTILES + wid) * _EPT
    nsl = pl.ds(base_n, _SL)
    own_deg = pl.ds((2 * _NCORES + cid) * _NPAD + base_n, _SL)
    mir_deg = pl.ds((2 * _NCORES + 1 - cid) * _NPAD + base_n, _SL)
    srcb = (srcb0, srcb1, srcb2)
    dstb = (dstb0, dstb1, dstb2)
    qs = (qs0, qs1)
    qd = (qd0, qd1)
    sem_ld = (sem_ld0, sem_ld1, sem_ld2)
    sem_g = (sem_g0, sem_g1)
    sem_s = (sem_s0, sem_s1)

    def _cross_handshake():
        pl.semaphore_signal(sem_x, 1, core_index=1 - cid)
        pl.semaphore_wait(sem_x, 1)

    pltpu.sync_copy(scale_hbm, scale_v)

    def _zero_loop(j, c):
        zeros_v[pl.ds(j * 16, 16)] = jnp.zeros((16,), jnp.float32)
        return c
    lax.fori_loop(0, _NV, _zero_loop, 0)

    def _ones_loop(j, c):
        qs0[pl.ds(j * 16, 16)] = jnp.ones((16,), jnp.float32)
        return c
    lax.fori_loop(0, _CE // 16, _ones_loop, 0)

    pltpu.sync_copy(zeros_v, ypos_sh.at[nsl])
    pltpu.sync_copy(zeros_v, yneg_sh.at[nsl])
    plsc.subcore_barrier()

    def _deg_loop(i, c):
        pltpu.sync_copy(dst_hbm.at[pl.ds(base_e + i * _CE, _CE)], dstb0)
        pltpu.sync_copy(qs0, ypos_sh.at[dstb0], add=True)
        return c
    lax.fori_loop(0, _NCHUNK, _deg_loop, 0)
    plsc.subcore_barrier()

    pltpu.sync_copy(ypos_sh.at[nsl], ytp)
    pltpu.sync_copy(ytp, xdiff_hbm.at[own_deg])
    pltpu.sync_copy(zeros_v, ypos_sh.at[nsl])
    _cross_handshake()
    pltpu.sync_copy(xdiff_hbm.at[mir_deg], ytn)
    pltpu.sync_copy(x_hbm.at[nsl], qs1.at[pl.ds(0, _SL)])

    def _init_loop(j, c):
        s = pl.ds(j * 16, 16)
        deg16 = ytp[s] + ytn[s] + 1.0
        dinv2_v[s] = 1.0 / deg16
        bits = lax.bitcast_convert_type(deg16, jnp.int32)
        y = lax.bitcast_convert_type(jnp.int32(0x5F3759DF) - (bits >> 1),
                                     jnp.float32)
        for _ in range(4):
            y = y * (1.5 - 0.5 * deg16 * y * y)
        sdeg_v[s] = deg16 * y
        p0 = qs1[s] * y
        pacc_v[s] = p0
        ytn[s] = p0
        return c
    lax.fori_loop(0, _NV, _init_loop, 0)

    pltpu.sync_copy(ytn, q_sh.at[nsl])

    def _ld_pair(i):
        bi = i % 3
        r = pl.ds(base_e + i * _CE, _CE)
        d1 = pltpu.make_async_copy(src_hbm.at[r], srcb[bi], sem_ld[bi])
        d2 = pltpu.make_async_copy(dst_hbm.at[r], dstb[bi], sem_ld[bi])
        return (d1, d2)

    def _fire_ld(i):
        d = _ld_pair(i)
        d[0].start()
        d[1].start()
        return d

    _fire_ld(0)
    _fire_ld(1)
    plsc.subcore_barrier()

    def _app_loop(a, c):
        sc16 = plsc.load_gather(scale_v, [jnp.full((16,), a, dtype=jnp.int32)])

        ld_desc = {0: _ld_pair(0), 1: _ld_pair(1)}
        sc_desc = {}
        for i in range(_NCHUNK):
            bi = i % 3
            bv = i % 2
            if i >= 2:
                sc_desc[i - 2][0].wait()
                sc_desc[i - 2][1].wait()
            if 2 <= i + 1 < _NCHUNK:
                ld_desc[i + 1] = _fire_ld(i + 1)
            ld_desc[i][0].wait()
            ld_desc[i][1].wait()
            g1 = pltpu.async_copy(q_sh.at[srcb[bi]], qs[bv], sem_g[bv])
            g2 = pltpu.async_copy(q_sh.at[dstb[bi]], qd[bv], sem_g[bv])
            g1.wait()
            g2.wait()
            s1 = pltpu.async_copy(qs[bv], ypos_sh.at[dstb[bi]], sem_s[bv],
                                  add=True)
            s2 = pltpu.async_copy(qd[bv], yneg_sh.at[srcb[bi]], sem_s[bv],
                                  add=True)
            sc_desc[i] = (s1, s2)
        for i in (_NCHUNK - 2, _NCHUNK - 1):
            sc_desc[i][0].wait()
            sc_desc[i][1].wait()
        _fire_ld(0)
        _fire_ld(1)
        plsc.subcore_barrier()

        da = pltpu.async_copy(ypos_sh.at[nsl], ytp, sem_p0)
        db = pltpu.async_copy(yneg_sh.at[nsl], ytn, sem_p1)
        da.wait()
        db.wait()
        z1 = pltpu.async_copy(zeros_v, ypos_sh.at[nsl], sem_p0)
        z2 = pltpu.async_copy(zeros_v, yneg_sh.at[nsl], sem_p1)

        def _diff_loop(j, cc):
            s = pl.ds(j * 16, 16)
            ytp[s] = ytp[s] - ytn[s]
            return cc
        lax.fori_loop(0, _NV, _diff_loop, 0)

        par = a % 2
        own_x = pl.ds((par * _NCORES + cid) * _NPAD + base_n, _SL)
        mir_x = pl.ds((par * _NCORES + 1 - cid) * _NPAD + base_n, _SL)
        xw = pltpu.async_copy(ytp, xdiff_hbm.at[own_x], sem_p2)
        xw.wait()
        _cross_handshake()
        xr = pltpu.async_copy(xdiff_hbm.at[mir_x], ytn, sem_p2)
        xr.wait()

        is_end16 = jnp.full((16,), a % _T, jnp.int32) == (_T - 1)

        def _pw_loop(j, cc):
            s = pl.ds(j * 16, 16)
            p16 = sc16 * dinv2_v[s] * (ytp[s] + ytn[s])
            acc16 = pacc_v[s] + p16
            pacc_v[s] = acc16
            ytp[s] = jnp.where(is_end16, acc16, p16)
            return cc
        lax.fori_loop(0, _NV, _pw_loop, 0)

        wq = pltpu.async_copy(ytp, q_sh.at[nsl], sem_p2)
        z1.wait()
        z2.wait()
        wq.wait()
        plsc.subcore_barrier()
        return c
    lax.fori_loop(0, _L * _T, _app_loop, 0)

    for i in (0, 1):
        d = _ld_pair(i)
        d[0].wait()
        d[1].wait()

    def _out_loop(j, c):
        s = pl.ds(j * 16, 16)
        ytp[s] = sdeg_v[s] * pacc_v[s]
        return c
    lax.fori_loop(0, _NV, _out_loop, 0)

    @pl.when(cid == 0)
    def _():
        pltpu.sync_copy(ytp, out_hbm.at[nsl])


_sc_call = functools.partial(
    pl.kernel,
    out_type=(jax.ShapeDtypeStruct((_NPAD,), jnp.float32),
              jax.ShapeDtypeStruct((3 * _NCORES * _NPAD,), jnp.float32)),
    mesh=plsc.VectorSubcoreMesh(
        core_axis_name="c", subcore_axis_name="s", num_cores=_NCORES),
    compiler_params=pltpu.CompilerParams(needs_layout_passes=False),
    scratch_types=[
        pltpu.VMEM_SHARED((_NPAD,), jnp.float32),
        pltpu.VMEM_SHARED((_NPAD,), jnp.float32),
        pltpu.VMEM_SHARED((_NPAD,), jnp.float32),
        pltpu.VMEM((_CE,), jnp.int32),
        pltpu.VMEM((_CE,), jnp.int32),
        pltpu.VMEM((_CE,), jnp.int32),
        pltpu.VMEM((_CE,), jnp.int32),
        pltpu.VMEM((_CE,), jnp.int32),
        pltpu.VMEM((_CE,), jnp.int32),
        pltpu.VMEM((_CE,), jnp.float32),
        pltpu.VMEM((_CE,), jnp.float32),
        pltpu.VMEM((_CE,), jnp.float32),
        pltpu.VMEM((_CE,), jnp.float32),
        pltpu.VMEM((_SL,), jnp.float32),
        pltpu.VMEM((_SL,), jnp.float32),
        pltpu.VMEM((_SL,), jnp.float32),
        pltpu.VMEM((_SL,), jnp.float32),
        pltpu.VMEM((_SL,), jnp.float32),
        pltpu.VMEM((_SL,), jnp.float32),
        pltpu.VMEM((128,), jnp.float32),
        pltpu.SemaphoreType.DMA,
        pltpu.SemaphoreType.DMA,
        pltpu.SemaphoreType.DMA,
        pltpu.SemaphoreType.DMA,
        pltpu.SemaphoreType.DMA,
        pltpu.SemaphoreType.DMA,
        pltpu.SemaphoreType.DMA,
        pltpu.SemaphoreType.DMA,
        pltpu.SemaphoreType.DMA,
        pltpu.SemaphoreType.DMA,
        pltpu.SemaphoreType.REGULAR,
    ],
)(_sc_body)


def kernel(x, edge_index, theta):
    xf = jnp.pad(x.reshape(_N), (0, _NPAD - _N))
    padi = jnp.full((_EPAD - _E,), _NPAD - 1, jnp.int32)
    src2 = jnp.concatenate([edge_index[0], padi])
    dst2 = jnp.concatenate([edge_index[1], padi])
    a_idx = jnp.arange(_L * _T)
    scales = theta[a_idx // _T] / (a_idx % _T + 1).astype(jnp.float32)
    scales = jnp.pad(scales, (0, 128 - _L * _T))
    out, _ = _sc_call(src2, dst2, xf, scales)
    return out[:_N].reshape(_N, 1, 1)
